# baseline (speedup 1.0000x reference)
import functools

import jax
import jax.numpy as jnp
from jax import lax
from jax.experimental import pallas as pl
from jax.experimental.pallas import tpu as pltpu
from jax.experimental.pallas import tpu_sc as plsc

_H = 512
_LANES = 16
_NCORES = 2
_NSUB = 16
_NW = _NCORES * _NSUB
_CH = _H // _LANES
_C = 16


def _tc_body(f0, f1, f2, f3, f4, f5, f6, f7, f8, w_ref, out_ref):
    w = w_ref[0, :][None, :]
    feats = [r[...] for r in (f0, f1, f2, f3, f4, f5, f6, f7, f8)]
    scores = [jnp.sum(f * w, axis=1, keepdims=True) for f in feats]
    m = scores[0]
    for s in scores[1:]:
        m = jnp.maximum(m, s)
    exps = [jnp.exp(s - m) for s in scores]
    denom = exps[0]
    for e in exps[1:]:
        denom = denom + e
    inv = 1.0 / denom
    acc = feats[0] * (exps[0] * inv)
    for i in range(1, 9):
        acc = acc + feats[i] * (exps[i] * inv)
    out_ref[...] = acc


def _tc_fuse(feats, w2, n_rows, blk):
    feat_spec = pl.BlockSpec((blk, _H), lambda i: (i, 0))
    return pl.pallas_call(
        _tc_body,
        grid=(n_rows // blk,),
        in_specs=[feat_spec] * 9 + [pl.BlockSpec((1, _H), lambda i: (0, 0))],
        out_specs=feat_spec,
        out_shape=jax.ShapeDtypeStruct((n_rows, _H), jnp.float32),
    )(*feats, w2)


def _make_sc(n_rows_sc, row0):
    rpw = n_rows_sc // _NW
    nch = rpw // _C
    mesh = plsc.VectorSubcoreMesh(core_axis_name="c", subcore_axis_name="s")
    scratch = ([pltpu.VMEM((_C, _H), jnp.float32) for _ in range(9)]
               + [pltpu.VMEM((_H,), jnp.float32),
                  pltpu.VMEM((_C, _H), jnp.float32)])

    @functools.partial(
        pl.kernel, mesh=mesh,
        out_type=jax.ShapeDtypeStruct((n_rows_sc, _H), jnp.float32),
        scratch_types=scratch,
        compiler_params=pltpu.CompilerParams(needs_layout_passes=False),
    )
    def sc_kernel(f0, f1, f2, f3, f4, f5, f6, f7, f8, w_hbm, out_hbm,
                  v0, v1, v2, v3, v4, v5, v6, v7, v8, wv, ov):
        fhs = (f0, f1, f2, f3, f4, f5, f6, f7, f8)
        fvs = (v0, v1, v2, v3, v4, v5, v6, v7, v8)
        wid = lax.axis_index("s") * _NCORES + lax.axis_index("c")
        pltpu.sync_copy(w_hbm, wv)
        wch = [wv[pl.ds(_LANES * k, _LANES)] for k in range(_CH)]
        base0 = row0 + wid * rpw

        def chunk_body(g, carry):
            base = base0 + g * _C
            for i in range(9):
                pltpu.sync_copy(fhs[i].at[pl.ds(base, _C), :], fvs[i])

            def row_body(r, c2):
                ss = []
                for i in range(9):
                    acc = fvs[i][r, pl.ds(0, _LANES)] * wch[0]
                    for k in range(1, _CH):
                        acc = acc + fvs[i][r, pl.ds(_LANES * k, _LANES)] * wch[k]
                    ss.append(jnp.sum(acc))
                m = ss[0]
                for s in ss[1:]:
                    m = jnp.maximum(m, s)
                es = [jnp.exp(jnp.full((_LANES,), s - m)) for s in ss]
                den = es[0]
                for e in es[1:]:
                    den = den + e
                atts = [e / den for e in es]
                for k in range(_CH):
                    acc = fvs[0][r, pl.ds(_LANES * k, _LANES)] * atts[0]
                    for i in range(1, 9):
                        acc = acc + fvs[i][r, pl.ds(_LANES * k, _LANES)] * atts[i]
                    ov[r, pl.ds(_LANES * k, _LANES)] = acc
                return c2
            lax.fori_loop(0, _C, row_body, 0)

            pltpu.sync_copy(ov, out_hbm.at[pl.ds(base - row0, _C), :])
            return carry
        lax.fori_loop(0, nch, chunk_body, 0)

    return sc_kernel


def kernel(feat_0, feat_1, feat_2, feat_3, feat_4, feat_5, feat_6, feat_7,
           feat_8, W):
    S, B, H = feat_0.shape
    R = S * B
    feats = [f.reshape(R, H) for f in (feat_0, feat_1, feat_2, feat_3, feat_4,
                                       feat_5, feat_6, feat_7, feat_8)]
    w2 = W.reshape(1, H)

    n_sc = 4096 if R >= 8192 else 0
    n_tc = R - n_sc

    tc_out = _tc_fuse(feats, w2, n_tc, min(512, n_tc))
    if n_sc == 0:
        return tc_out.reshape(S, B, H)

    sc_out = _make_sc(n_sc, n_tc)(*feats, W)
    full = jnp.concatenate([tc_out, sc_out], axis=0)
    return full.reshape(S, B, H)


# speedup vs baseline: 1.0145x; 1.0145x over previous
import functools

import jax
import jax.numpy as jnp
from jax import lax
from jax.experimental import pallas as pl
from jax.experimental.pallas import tpu as pltpu
from jax.experimental.pallas import tpu_sc as plsc

_H = 512
_LANES = 16
_NCORES = 2
_NSUB = 16
_NW = _NCORES * _NSUB
_CH = _H // _LANES
_C = 16


def _tc_body(f0, f1, f2, f3, f4, f5, f6, f7, f8, w_ref, out_ref):
    w = w_ref[0, :][None, :]
    feats = [r[...] for r in (f0, f1, f2, f3, f4, f5, f6, f7, f8)]
    scores = [jnp.sum(f * w, axis=1, keepdims=True) for f in feats]
    m = scores[0]
    for s in scores[1:]:
        m = jnp.maximum(m, s)
    exps = [jnp.exp(s - m) for s in scores]
    denom = exps[0]
    for e in exps[1:]:
        denom = denom + e
    inv = 1.0 / denom
    acc = feats[0] * (exps[0] * inv)
    for i in range(1, 9):
        acc = acc + feats[i] * (exps[i] * inv)
    out_ref[...] = acc


def _tc_fuse(feats, w2, n_rows, blk):
    feat_spec = pl.BlockSpec((blk, _H), lambda i: (i, 0))
    return pl.pallas_call(
        _tc_body,
        grid=(n_rows // blk,),
        in_specs=[feat_spec] * 9 + [pl.BlockSpec((1, _H), lambda i: (0, 0))],
        out_specs=feat_spec,
        out_shape=jax.ShapeDtypeStruct((n_rows, _H), jnp.float32),
    )(*feats, w2)


def _make_sc(n_rows_sc, row0):
    rpw = n_rows_sc // _NW
    nch = rpw // _C
    mesh = plsc.VectorSubcoreMesh(core_axis_name="c", subcore_axis_name="s")
    scratch = ([pltpu.VMEM((_C, _H), jnp.float32) for _ in range(9)]
               + [pltpu.VMEM((_H,), jnp.float32),
                  pltpu.VMEM((_C, _H), jnp.float32)])

    @functools.partial(
        pl.kernel, mesh=mesh,
        out_type=jax.ShapeDtypeStruct((n_rows_sc, _H), jnp.float32),
        scratch_types=scratch,
        compiler_params=pltpu.CompilerParams(needs_layout_passes=False),
    )
    def sc_kernel(f0, f1, f2, f3, f4, f5, f6, f7, f8, w_hbm, out_hbm,
                  v0, v1, v2, v3, v4, v5, v6, v7, v8, wv, ov):
        fhs = (f0, f1, f2, f3, f4, f5, f6, f7, f8)
        fvs = (v0, v1, v2, v3, v4, v5, v6, v7, v8)
        wid = lax.axis_index("s") * _NCORES + lax.axis_index("c")
        pltpu.sync_copy(w_hbm, wv)
        wch = [wv[pl.ds(_LANES * k, _LANES)] for k in range(_CH)]
        base0 = row0 + wid * rpw

        def chunk_body(g, carry):
            base = base0 + g * _C
            for i in range(9):
                pltpu.sync_copy(fhs[i].at[pl.ds(base, _C), :], fvs[i])

            def row_body(r, c2):
                ss = []
                for i in range(9):
                    accs = [fvs[i][r, pl.ds(_LANES * a, _LANES)] * wch[a]
                            for a in range(4)]
                    for k in range(4, _CH):
                        a = k % 4
                        accs[a] = accs[a] + (
                            fvs[i][r, pl.ds(_LANES * k, _LANES)] * wch[k])
                    acc = (accs[0] + accs[1]) + (accs[2] + accs[3])
                    ss.append(jnp.sum(acc))
                m = ss[0]
                for s in ss[1:]:
                    m = jnp.maximum(m, s)
                es = [jnp.exp(jnp.full((_LANES,), s - m)) for s in ss]
                den = es[0]
                for e in es[1:]:
                    den = den + e
                atts = [e / den for e in es]
                for k in range(_CH):
                    a0 = fvs[0][r, pl.ds(_LANES * k, _LANES)] * atts[0]
                    a1 = fvs[1][r, pl.ds(_LANES * k, _LANES)] * atts[1]
                    a2 = fvs[2][r, pl.ds(_LANES * k, _LANES)] * atts[2]
                    for i in range(3, 9):
                        if i % 3 == 0:
                            a0 = a0 + fvs[i][r, pl.ds(_LANES * k, _LANES)] * atts[i]
                        elif i % 3 == 1:
                            a1 = a1 + fvs[i][r, pl.ds(_LANES * k, _LANES)] * atts[i]
                        else:
                            a2 = a2 + fvs[i][r, pl.ds(_LANES * k, _LANES)] * atts[i]
                    ov[r, pl.ds(_LANES * k, _LANES)] = a0 + (a1 + a2)
                return c2
            lax.fori_loop(0, _C, row_body, 0)

            pltpu.sync_copy(ov, out_hbm.at[pl.ds(base - row0, _C), :])
            return carry
        lax.fori_loop(0, nch, chunk_body, 0)

    return sc_kernel


def kernel(feat_0, feat_1, feat_2, feat_3, feat_4, feat_5, feat_6, feat_7,
           feat_8, W):
    S, B, H = feat_0.shape
    R = S * B
    feats = [f.reshape(R, H) for f in (feat_0, feat_1, feat_2, feat_3, feat_4,
                                       feat_5, feat_6, feat_7, feat_8)]
    w2 = W.reshape(1, H)

    n_sc = 4096 if R >= 8192 else 0
    n_tc = R - n_sc

    tc_out = _tc_fuse(feats, w2, n_tc, min(512, n_tc))
    if n_sc == 0:
        return tc_out.reshape(S, B, H)

    sc_out = _make_sc(n_sc, n_tc)(*feats, W)
    full = jnp.concatenate([tc_out, sc_out], axis=0)
    return full.reshape(S, B, H)
